# baseline (speedup 1.0000x reference)
import functools

import jax
import jax.numpy as jnp
from jax.experimental import pallas as pl
from jax.experimental.pallas import tpu as pltpu

_TOP_K = 8
_SPARSE_EPS = 1e-4

_BD = 64


def _routing_weights(c):
    B, E = c.shape
    col = jax.lax.broadcasted_iota(jnp.int32, (B, E), 1)
    rank = jnp.zeros((B, E), dtype=jnp.int32)
    for j in range(E):
        cj = c[:, j : j + 1]
        beats = (cj > c) | ((cj == c) & (j < col))
        rank = rank + beats.astype(jnp.int32)
    cm = jnp.where(rank < _TOP_K, c, 0.0)
    m = jnp.max(cm, axis=1, keepdims=True)
    ex = jnp.exp(cm - m)
    p = ex / jnp.sum(ex, axis=1, keepdims=True)
    p = jnp.where(p < _SPARSE_EPS, 0.0, p)
    return p.astype(jnp.bfloat16)


def _ensemble_kernel(cfg_ref, attr_ref, out_ref, p_bf16, *, bd):
    @pl.when(pl.program_id(0) == 0)
    def _():
        p_bf16[...] = _routing_weights(cfg_ref[...])

    p = p_bf16[...]
    for d in range(bd):
        a = attr_ref[:, d, :].astype(jnp.bfloat16)
        out_ref[:, d, :] = jnp.dot(p, a, preferred_element_type=jnp.float32)


def kernel(configuration, kernel):
    B, E = configuration.shape
    E2, d1, d2 = kernel.shape
    n_blocks = d1 // _BD

    return pl.pallas_call(
        functools.partial(_ensemble_kernel, bd=_BD),
        grid=(n_blocks,),
        in_specs=[
            pl.BlockSpec((B, E), lambda i: (0, 0)),
            pl.BlockSpec((E2, _BD, d2), lambda i: (0, i, 0)),
        ],
        out_specs=pl.BlockSpec((B, _BD, d2), lambda i: (0, i, 0)),
        out_shape=jax.ShapeDtypeStruct((B, d1, d2), jnp.float32),
        scratch_shapes=[
            pltpu.VMEM((B, E), jnp.bfloat16),
        ],
        compiler_params=pltpu.CompilerParams(
            dimension_semantics=("arbitrary",),
        ),
    )(configuration, kernel)


# speedup vs baseline: 1.0211x; 1.0211x over previous
import functools

import jax
import jax.numpy as jnp
from jax.experimental import pallas as pl
from jax.experimental.pallas import tpu as pltpu

_TOP_K = 8
_SPARSE_EPS = 1e-4

_BD = 16


def _routing_weights(c):
    B, E = c.shape
    col = jax.lax.broadcasted_iota(jnp.int32, (B, E), 1)
    rank = jnp.zeros((B, E), dtype=jnp.int32)
    for j in range(E):
        cj = c[:, j : j + 1]
        beats = (cj > c) | ((cj == c) & (j < col))
        rank = rank + beats.astype(jnp.int32)
    cm = jnp.where(rank < _TOP_K, c, 0.0)
    m = jnp.max(cm, axis=1, keepdims=True)
    ex = jnp.exp(cm - m)
    p = ex / jnp.sum(ex, axis=1, keepdims=True)
    p = jnp.where(p < _SPARSE_EPS, 0.0, p)
    return p.astype(jnp.bfloat16)


def _ensemble_kernel(cfg_ref, attr_ref, out_ref, p_bf16, *, bd):
    @pl.when(pl.program_id(0) == 0)
    def _():
        p_bf16[...] = _routing_weights(cfg_ref[...])

    p = p_bf16[...]
    for d in range(bd):
        a = attr_ref[:, d, :].astype(jnp.bfloat16)
        out_ref[:, d, :] = jnp.dot(p, a, preferred_element_type=jnp.float32)


def kernel(configuration, kernel):
    B, E = configuration.shape
    E2, d1, d2 = kernel.shape
    n_blocks = d1 // _BD

    return pl.pallas_call(
        functools.partial(_ensemble_kernel, bd=_BD),
        grid=(n_blocks,),
        in_specs=[
            pl.BlockSpec((B, E), lambda i: (0, 0)),
            pl.BlockSpec((E2, _BD, d2), lambda i: (0, i, 0)),
        ],
        out_specs=pl.BlockSpec((B, _BD, d2), lambda i: (0, i, 0)),
        out_shape=jax.ShapeDtypeStruct((B, d1, d2), jnp.float32),
        scratch_shapes=[
            pltpu.VMEM((B, E), jnp.bfloat16),
        ],
        compiler_params=pltpu.CompilerParams(
            dimension_semantics=("arbitrary",),
        ),
    )(configuration, kernel)


# speedup vs baseline: 1.6945x; 1.6595x over previous
import functools

import jax
import jax.numpy as jnp
from jax.experimental import pallas as pl
from jax.experimental.pallas import tpu as pltpu

_TOP_K = 8
_SPARSE_EPS = 1e-4

_BD = 64
_NW = 4


def _routing_weights(c):
    B, E = c.shape
    col = jax.lax.broadcasted_iota(jnp.int32, (B, E), 1)
    rank = jnp.zeros((B, E), dtype=jnp.int32)
    for j in range(E):
        cj = c[:, j : j + 1]
        beats = (cj > c) | ((cj == c) & (j < col))
        rank = rank + beats.astype(jnp.int32)
    cm = jnp.where(rank < _TOP_K, c, 0.0)
    m = jnp.max(cm, axis=1, keepdims=True)
    ex = jnp.exp(cm - m)
    p = ex / jnp.sum(ex, axis=1, keepdims=True)
    return jnp.where(p < _SPARSE_EPS, 0.0, p)


def _ensemble_kernel(cfg_ref, attr_ref, out_hbm, p_f32, obuf, osem, *,
                     bd, n_blocks, nw, rb):
    i = pl.program_id(0)

    @pl.when(i == 0)
    def _():
        p_f32[...] = _routing_weights(cfg_ref[...])

    def copies(step, slot):
        return [
            pltpu.make_async_copy(
                obuf.at[slot, pl.ds(w * rb, rb)],
                out_hbm.at[pl.ds(w * rb, rb), pl.ds(step * bd, bd), :],
                osem.at[slot, w])
            for w in range(nw)
        ]

    slot = jax.lax.rem(i, 2)

    @pl.when(i >= 2)
    def _():
        for c in copies(i - 2, slot):
            c.wait()

    p = p_f32[...]
    for d in range(bd):
        obuf[slot, :, d, :] = jnp.dot(p, attr_ref[:, d, :],
                                      preferred_element_type=jnp.float32)
    for c in copies(i, slot):
        c.start()

    @pl.when(i == n_blocks - 1)
    def _():
        for c in copies(i - 1, 1 - slot):
            c.wait()
        for c in copies(i, slot):
            c.wait()


def kernel(configuration, kernel):
    B, E = configuration.shape
    E2, d1, d2 = kernel.shape
    n_blocks = d1 // _BD
    rb = B // _NW

    return pl.pallas_call(
        functools.partial(_ensemble_kernel, bd=_BD, n_blocks=n_blocks,
                          nw=_NW, rb=rb),
        grid=(n_blocks,),
        in_specs=[
            pl.BlockSpec((B, E), lambda i: (0, 0)),
            pl.BlockSpec((E2, _BD, d2), lambda i: (0, i, 0)),
        ],
        out_specs=pl.BlockSpec(memory_space=pltpu.MemorySpace.HBM),
        out_shape=jax.ShapeDtypeStruct((B, d1, d2), jnp.float32),
        scratch_shapes=[
            pltpu.VMEM((B, E), jnp.float32),
            pltpu.VMEM((2, B, _BD, d2), jnp.float32),
            pltpu.SemaphoreType.DMA((2, _NW)),
        ],
        compiler_params=pltpu.CompilerParams(
            dimension_semantics=("arbitrary",),
        ),
    )(configuration, kernel)


# speedup vs baseline: 1.7078x; 1.0079x over previous
import functools

import jax
import jax.numpy as jnp
from jax.experimental import pallas as pl
from jax.experimental.pallas import tpu as pltpu

_TOP_K = 8
_SPARSE_EPS = 1e-4

_BD = 64


def _routing_weights(c):
    B, E = c.shape
    col = jax.lax.broadcasted_iota(jnp.int32, (B, E), 1)
    rank = jnp.zeros((B, E), dtype=jnp.int32)
    for j in range(E):
        cj = c[:, j : j + 1]
        beats = (cj > c) | ((cj == c) & (j < col))
        rank = rank + beats.astype(jnp.int32)
    cm = jnp.where(rank < _TOP_K, c, 0.0)
    m = jnp.max(cm, axis=1, keepdims=True)
    ex = jnp.exp(cm - m)
    p = ex / jnp.sum(ex, axis=1, keepdims=True)
    return jnp.where(p < _SPARSE_EPS, 0.0, p)


def _ensemble_kernel(cfg_ref, attr_ref, out_ref, p_f32, *, bd):
    @pl.when(pl.program_id(0) == 0)
    def _():
        p_f32[...] = _routing_weights(cfg_ref[...])

    p = p_f32[...]
    for d in range(bd):
        out_ref[:, d, :] = jnp.dot(p, attr_ref[:, d, :],
                                   preferred_element_type=jnp.float32)


def kernel(configuration, kernel):
    B, E = configuration.shape
    E2, d1, d2 = kernel.shape
    n_blocks = d1 // _BD

    return pl.pallas_call(
        functools.partial(_ensemble_kernel, bd=_BD),
        grid=(n_blocks,),
        in_specs=[
            pl.BlockSpec((B, E), lambda i: (0, 0)),
            pl.BlockSpec((E2, _BD, d2), lambda i: (0, i, 0)),
        ],
        out_specs=pl.BlockSpec((B, _BD, d2), lambda i: (0, i, 0)),
        out_shape=jax.ShapeDtypeStruct((B, d1, d2), jnp.float32),
        scratch_shapes=[
            pltpu.VMEM((B, E), jnp.float32),
        ],
        compiler_params=pltpu.CompilerParams(
            dimension_semantics=("parallel",),
        ),
    )(configuration, kernel)
